# Initial kernel scaffold; baseline (speedup 1.0000x reference)
import jax
import jax.numpy as jnp
from jax.experimental import pallas as pl


def kernel(x, edge_index, edge_values, gamma, beta, ln_gamma, ln_beta, W, b):
    raise NotImplementedError("write your pallas kernel here")


# speedup vs baseline: 4.4577x; 4.4577x over previous
import functools

import jax
import jax.numpy as jnp
from jax import lax
from jax.experimental import pallas as pl
from jax.experimental.pallas import tpu as pltpu
from jax.experimental.pallas import tpu_sc as plsc

N, D, E = 10000, 128, 320000
NC, NS = 2, 16
NW = NC * NS
EPT = E // NW
K = 80
NCHUNK = EPT // K
CPR = 80
NCP = N // CPR
LANES = 16


def _head_body(x_ref, w_ref, g_ref, b_ref, o_ref):
    xb = x_ref[...]
    mu = jnp.mean(xb, axis=-1, keepdims=True)
    var = jnp.mean((xb - mu) * (xb - mu), axis=-1, keepdims=True)
    h = (xb - mu) * lax.rsqrt(var + 1e-5) * g_ref[...] + b_ref[...]
    h = h * jax.nn.sigmoid(h)
    o_ref[...] = lax.dot_general(
        h, w_ref[...], (((1,), (1,)), ((), ())),
        preferred_element_type=jnp.float32)


def _head(x, W, ln_gamma, ln_beta):
    BR = 2000
    grid = (N // BR,)
    return pl.pallas_call(
        _head_body,
        grid=grid,
        in_specs=[
            pl.BlockSpec((BR, D), lambda i: (i, 0)),
            pl.BlockSpec((D, D), lambda i: (0, 0)),
            pl.BlockSpec((1, D), lambda i: (0, 0)),
            pl.BlockSpec((1, D), lambda i: (0, 0)),
        ],
        out_specs=pl.BlockSpec((BR, D), lambda i: (i, 0)),
        out_shape=jax.ShapeDtypeStruct((N, D), jnp.float32),
    )(x, W, ln_gamma.reshape(1, D), ln_beta.reshape(1, D))


def _sc_body(h_hbm, row_hbm, col_hbm, val_hbm, out_hbm,
             z_sh, rows_v, ridx_v, cidx_v, val_v, sem):
    c = lax.axis_index("c")
    s = lax.axis_index("s")

    def _zero_row(i, _):
        for j in range(D // LANES):
            rows_v[i, pl.ds(j * LANES, LANES)] = jnp.zeros((LANES,), jnp.float32)
        return ()
    lax.fori_loop(0, CPR, _zero_row, ())
    nz = jnp.where(s < NCP % NS, NCP // NS + 1, NCP // NS)

    def _zero_chunk(q, _):
        pltpu.sync_copy(rows_v, z_sh.at[pl.ds((q * NS + s) * CPR, CPR)])
        return ()
    lax.fori_loop(0, nz, _zero_chunk, ())
    plsc.subcore_barrier()

    base = (c * NS + s) * EPT

    def _chunk(k, _):
        off = base + k * K
        pltpu.sync_copy(row_hbm.at[pl.ds(off, K)], ridx_v)
        pltpu.sync_copy(col_hbm.at[pl.ds(off, K)], cidx_v)
        pltpu.sync_copy(val_hbm.at[pl.ds(off, K)], val_v)
        pltpu.async_copy(h_hbm.at[cidx_v], rows_v, sem).wait()

        def _group(g, _):
            v16 = val_v[pl.ds(g * LANES, LANES)]
            for i in range(LANES):
                sp = lax.gather(
                    v16, jnp.full((LANES, 1), i, jnp.int32),
                    lax.GatherDimensionNumbers(
                        offset_dims=(), collapsed_slice_dims=(0,),
                        start_index_map=(0,)),
                    (1,), mode=lax.GatherScatterMode.PROMISE_IN_BOUNDS)
                e = g * LANES + i
                for j in range(D // LANES):
                    sl = pl.ds(j * LANES, LANES)
                    rows_v[e, sl] = rows_v[e, sl] * sp
            return ()
        lax.fori_loop(0, K // LANES, _group, ())

        pltpu.sync_copy(rows_v, z_sh.at[ridx_v], add=True)
        return ()
    lax.fori_loop(0, NCHUNK, _chunk, ())

    plsc.subcore_barrier()

    def _out_chunk(q, _):
        r0 = (q * NS + s) * CPR
        pltpu.sync_copy(z_sh.at[pl.ds(r0, CPR)], rows_v)
        pltpu.sync_copy(rows_v, out_hbm.at[c, pl.ds(r0, CPR)])
        return ()
    lax.fori_loop(0, nz, _out_chunk, ())


def _sc_spmm(h2, row, col, val):
    mesh = plsc.VectorSubcoreMesh(core_axis_name="c", subcore_axis_name="s")
    f = pl.kernel(
        _sc_body,
        out_type=jax.ShapeDtypeStruct((NC, N, D), jnp.float32),
        mesh=mesh,
        scratch_types=[
            pltpu.VMEM_SHARED((N, D), jnp.float32),
            pltpu.VMEM((K, D), jnp.float32),
            pltpu.VMEM((K,), jnp.int32),
            pltpu.VMEM((K,), jnp.int32),
            pltpu.VMEM((K,), jnp.float32),
            pltpu.SemaphoreType.DMA,
        ],
    )
    return f(h2, row, col, val)


def _tail_body(x_ref, z0_ref, z1_ref, b_ref, g_ref, be_ref, o_ref):
    z = z0_ref[...] + z1_ref[...] + b_ref[...]
    o_ref[...] = x_ref[...] + z * g_ref[...] + be_ref[...]


def _tail(x, z0, z1, b, gamma, beta):
    BR = 2000
    grid = (N // BR,)
    blk = pl.BlockSpec((BR, D), lambda i: (i, 0))
    vec = pl.BlockSpec((1, D), lambda i: (0, 0))
    return pl.pallas_call(
        _tail_body,
        grid=grid,
        in_specs=[blk, blk, blk, vec, vec, vec],
        out_specs=blk,
        out_shape=jax.ShapeDtypeStruct((N, D), jnp.float32),
    )(x, z0, z1, b.reshape(1, D), gamma.reshape(1, D), beta.reshape(1, D))


@jax.jit
def kernel(x, edge_index, edge_values, gamma, beta, ln_gamma, ln_beta, W, b):
    h2 = _head(x, W, ln_gamma, ln_beta)
    row = edge_index[0].astype(jnp.int32)
    col = edge_index[1].astype(jnp.int32)
    zp = _sc_spmm(h2, row, col, edge_values)
    return _tail(x, zp[0], zp[1], b, gamma, beta)
